# Initial kernel scaffold; baseline (speedup 1.0000x reference)
import jax
import jax.numpy as jnp
from jax.experimental import pallas as pl


def kernel(token_ids, weight):
    raise NotImplementedError("write your pallas kernel here")


# speedup vs baseline: 3.3340x; 3.3340x over previous
import functools

import jax
import jax.numpy as jnp
from jax import lax
from jax.experimental import pallas as pl
from jax.experimental.pallas import tpu as pltpu
from jax.experimental.pallas import tpu_sc as plsc

_NUM_CORES = 2
_NUM_SUBCORES = 16
_NUM_WORKERS = _NUM_CORES * _NUM_SUBCORES


@functools.lru_cache(maxsize=None)
def _make_gather(B: int, V: int, D: int):
    assert B % _NUM_WORKERS == 0
    b_per_w = B // _NUM_WORKERS
    chunk = 320
    assert b_per_w % chunk == 0 and chunk % 8 == 0
    n_chunks = b_per_w // chunk

    mesh = plsc.VectorSubcoreMesh(
        core_axis_name="c", subcore_axis_name="s", num_cores=_NUM_CORES
    )

    @functools.partial(
        pl.kernel,
        mesh=mesh,
        out_type=jax.ShapeDtypeStruct((B, D), jnp.float32),
        scratch_types=[
            pltpu.VMEM((b_per_w,), jnp.int32),
            pltpu.VMEM((chunk, D), jnp.float32),
            pltpu.VMEM((chunk, D), jnp.float32),
            pltpu.SemaphoreType.DMA,
            pltpu.SemaphoreType.DMA,
        ],
    )
    def gather_kernel(idx_hbm, table_hbm, out_hbm, idx_v, buf0, buf1, sem0, sem1):
        wid = lax.axis_index("s") * _NUM_CORES + lax.axis_index("c")
        base = wid * b_per_w
        pltpu.sync_copy(idx_hbm.at[pl.ds(base, b_per_w)], idx_v)

        bufs = (buf0, buf1)
        sems = (sem0, sem1)

        def start(g):
            return pltpu.async_copy(
                table_hbm.at[idx_v.at[pl.ds(g * chunk, chunk)]],
                bufs[g % 2],
                sems[g % 2],
            )

        pending = start(0)
        for g in range(n_chunks):
            nxt = g + 1
            nxt_pending = start(nxt) if nxt < n_chunks else None
            pending.wait()
            pltpu.sync_copy(bufs[g % 2], out_hbm.at[pl.ds(base + g * chunk, chunk)])
            pending = nxt_pending

    return gather_kernel


def kernel(token_ids, weight):
    S0, S1 = token_ids.shape
    V, D = weight.shape
    idx = token_ids.reshape(-1).astype(jnp.int32)
    out = _make_gather(idx.shape[0], V, D)(idx, weight)
    return out.reshape(S0, S1, D)


# speedup vs baseline: 3.3350x; 1.0003x over previous
import functools

import jax
import jax.numpy as jnp
from jax import lax
from jax.experimental import pallas as pl
from jax.experimental.pallas import tpu as pltpu
from jax.experimental.pallas import tpu_sc as plsc

_NUM_CORES = 2
_NUM_SUBCORES = 16
_NUM_WORKERS = _NUM_CORES * _NUM_SUBCORES
_NBUF = 4


@functools.lru_cache(maxsize=None)
def _make_gather(B: int, V: int, D: int):
    assert B % _NUM_WORKERS == 0
    b_per_w = B // _NUM_WORKERS
    chunk = 160
    assert b_per_w % (chunk * _NBUF) == 0 and chunk % 8 == 0
    n_chunks = b_per_w // chunk
    n_outer = n_chunks // _NBUF

    mesh = plsc.VectorSubcoreMesh(
        core_axis_name="c", subcore_axis_name="s", num_cores=_NUM_CORES
    )

    @functools.partial(
        pl.kernel,
        mesh=mesh,
        out_type=jax.ShapeDtypeStruct((B, D), jnp.float32),
        scratch_types=[
            pltpu.VMEM((b_per_w,), jnp.int32),
            [pltpu.VMEM((chunk, D), jnp.float32) for _ in range(_NBUF)],
            [pltpu.SemaphoreType.DMA for _ in range(_NBUF)],
            [pltpu.SemaphoreType.DMA for _ in range(_NBUF)],
        ],
    )
    def gather_kernel(idx_hbm, table_hbm, out_hbm, idx_v, bufs, gsems, wsems):
        wid = lax.axis_index("s") * _NUM_CORES + lax.axis_index("c")
        base = wid * b_per_w
        pltpu.sync_copy(idx_hbm.at[pl.ds(base, b_per_w)], idx_v)

        def startg(c, b):
            pltpu.async_copy(
                table_hbm.at[idx_v.at[pl.ds(c * chunk, chunk)]], bufs[b], gsems[b]
            )

        def waitg(b):
            pltpu.make_async_copy(
                table_hbm.at[idx_v.at[pl.ds(0, chunk)]], bufs[b], gsems[b]
            ).wait()

        def startw(c, b):
            pltpu.async_copy(
                bufs[b], out_hbm.at[pl.ds(base + c * chunk, chunk)], wsems[b]
            )

        def waitw(b):
            pltpu.make_async_copy(
                bufs[b], out_hbm.at[pl.ds(base, chunk)], wsems[b]
            ).wait()

        for b in range(_NBUF - 1):
            startg(b, b)
        waitg(0)
        startw(0, 0)
        startg(_NBUF - 1, _NBUF - 1)
        for c in range(1, _NBUF):
            b = c % _NBUF
            waitg(b)
            startw(c, b)
            waitw((b - 1) % _NBUF)
            startg(c + _NBUF - 1, (b - 1) % _NBUF)

        @pl.loop(1, n_outer - 1)
        def _steady(o):
            c0 = o * _NBUF
            for b in range(_NBUF):
                waitg(b)
                startw(c0 + b, b)
                waitw((b - 1) % _NBUF)
                startg(c0 + b + _NBUF - 1, (b - 1) % _NBUF)

        c0 = (n_outer - 1) * _NBUF
        for b in range(_NBUF):
            waitg(b)
            startw(c0 + b, b)
            waitw((b - 1) % _NBUF)
            if b == 0:
                startg(c0 + b + _NBUF - 1, (b - 1) % _NBUF)
        waitw(_NBUF - 1)

    return gather_kernel


def kernel(token_ids, weight):
    S0, S1 = token_ids.shape
    V, D = weight.shape
    idx = token_ids.reshape(-1).astype(jnp.int32)
    out = _make_gather(idx.shape[0], V, D)(idx, weight)
    return out.reshape(S0, S1, D)


# speedup vs baseline: 5.8902x; 1.7662x over previous
import functools

import jax
import jax.numpy as jnp
from jax import lax
from jax.experimental import pallas as pl
from jax.experimental.pallas import tpu as pltpu
from jax.experimental.pallas import tpu_sc as plsc

_NUM_CORES = 2
_NUM_SUBCORES = 16
_NUM_WORKERS = _NUM_CORES * _NUM_SUBCORES
_NBUF = 4


@functools.lru_cache(maxsize=None)
def _make_gather(S0: int, S1: int, V: int, D: int):
    assert S0 % _NUM_WORKERS == 0
    s_per_w = S0 // _NUM_WORKERS
    b_per_w = s_per_w * S1
    chunk_s = 4
    chunk = chunk_s * S1
    assert s_per_w % (chunk_s * _NBUF) == 0 and chunk % 8 == 0
    n_chunks = s_per_w // chunk_s
    n_outer = n_chunks // _NBUF

    mesh = plsc.VectorSubcoreMesh(
        core_axis_name="c", subcore_axis_name="s", num_cores=_NUM_CORES
    )

    @functools.partial(
        pl.kernel,
        mesh=mesh,
        out_type=jax.ShapeDtypeStruct((S0, S1, D), jnp.float32),
        compiler_params=pltpu.CompilerParams(use_tc_tiling_on_sc=True),
        scratch_types=[
            pltpu.VMEM((b_per_w,), jnp.int32),
            [pltpu.VMEM((chunk, D), jnp.float32) for _ in range(_NBUF)],
            [pltpu.SemaphoreType.DMA for _ in range(_NBUF)],
            [pltpu.SemaphoreType.DMA for _ in range(_NBUF)],
        ],
    )
    def gather_kernel(idx_hbm, table_hbm, out_hbm, idx_v, bufs, gsems, wsems):
        wid = lax.axis_index("s") * _NUM_CORES + lax.axis_index("c")
        base = wid * b_per_w
        sbase = wid * s_per_w
        pltpu.sync_copy(idx_hbm.at[pl.ds(base, b_per_w)], idx_v)

        def startg(c, b):
            pltpu.async_copy(
                table_hbm.at[idx_v.at[pl.ds(c * chunk, chunk)]], bufs[b], gsems[b]
            )

        def waitg(b):
            pltpu.make_async_copy(
                table_hbm.at[idx_v.at[pl.ds(0, chunk)]], bufs[b], gsems[b]
            ).wait()

        def startw(c, b):
            for j in range(chunk_s):
                pltpu.async_copy(
                    bufs[b].at[pl.ds(j * S1, S1)],
                    out_hbm.at[sbase + c * chunk_s + j],
                    wsems[b],
                )

        def waitw(b):
            for j in range(chunk_s):
                pltpu.make_async_copy(
                    bufs[b].at[pl.ds(j * S1, S1)], out_hbm.at[sbase], wsems[b]
                ).wait()

        for b in range(_NBUF - 1):
            startg(b, b)
        waitg(0)
        startw(0, 0)
        startg(_NBUF - 1, _NBUF - 1)
        for c in range(1, _NBUF):
            b = c % _NBUF
            waitg(b)
            startw(c, b)
            waitw((b - 1) % _NBUF)
            startg(c + _NBUF - 1, (b - 1) % _NBUF)

        @pl.loop(1, n_outer - 1)
        def _steady(o):
            c0 = o * _NBUF
            for b in range(_NBUF):
                waitg(b)
                startw(c0 + b, b)
                waitw((b - 1) % _NBUF)
                startg(c0 + b + _NBUF - 1, (b - 1) % _NBUF)

        c0 = (n_outer - 1) * _NBUF
        for b in range(_NBUF):
            waitg(b)
            startw(c0 + b, b)
            waitw((b - 1) % _NBUF)
            if b == 0:
                startg(c0 + b + _NBUF - 1, (b - 1) % _NBUF)
        waitw(_NBUF - 1)

    return gather_kernel


def kernel(token_ids, weight):
    S0, S1 = token_ids.shape
    V, D = weight.shape
    idx = token_ids.reshape(-1).astype(jnp.int32)
    return _make_gather(S0, S1, V, D)(idx, weight)


# speedup vs baseline: 5.9070x; 1.0029x over previous
import functools

import jax
import jax.numpy as jnp
from jax import lax
from jax.experimental import pallas as pl
from jax.experimental.pallas import tpu as pltpu
from jax.experimental.pallas import tpu_sc as plsc

_NUM_CORES = 2
_NUM_SUBCORES = 16
_NUM_WORKERS = _NUM_CORES * _NUM_SUBCORES
_NBUF = 4


@functools.lru_cache(maxsize=None)
def _make_gather(S0: int, S1: int, V: int, D: int):
    assert S0 % _NUM_WORKERS == 0
    s_per_w = S0 // _NUM_WORKERS
    b_per_w = s_per_w * S1
    chunk_s = 4
    chunk = chunk_s * S1
    assert s_per_w % (chunk_s * _NBUF) == 0 and chunk % 8 == 0
    n_chunks = s_per_w // chunk_s
    n_outer = n_chunks // _NBUF

    mesh = plsc.VectorSubcoreMesh(
        core_axis_name="c", subcore_axis_name="s", num_cores=_NUM_CORES
    )

    @functools.partial(
        pl.kernel,
        mesh=mesh,
        out_type=jax.ShapeDtypeStruct((S0, S1, D), jnp.float32),
        compiler_params=pltpu.CompilerParams(
            use_tc_tiling_on_sc=True, needs_layout_passes=True
        ),
        scratch_types=[
            pltpu.VMEM((b_per_w,), jnp.int32),
            [pltpu.VMEM((chunk, D), jnp.float32) for _ in range(_NBUF)],
            [pltpu.SemaphoreType.DMA for _ in range(_NBUF)],
            [pltpu.SemaphoreType.DMA for _ in range(_NBUF)],
        ],
    )
    def gather_kernel(idx_hbm, table_hbm, out_hbm, idx_v, bufs, gsems, wsems):
        wid = lax.axis_index("s") * _NUM_CORES + lax.axis_index("c")
        base = wid * b_per_w
        sbase = wid * s_per_w
        pltpu.sync_copy(idx_hbm.at[pl.ds(base, b_per_w)], idx_v)

        def startg(c, b):
            pltpu.async_copy(
                table_hbm.at[idx_v.at[pl.ds(c * chunk, chunk)]], bufs[b], gsems[b]
            )

        def waitg(b):
            pltpu.make_async_copy(
                table_hbm.at[idx_v.at[pl.ds(0, chunk)]], bufs[b], gsems[b]
            ).wait()

        def startw(c, b):
            for j in range(chunk_s):
                pltpu.async_copy(
                    bufs[b].at[pl.ds(j * S1, S1)],
                    out_hbm.at[sbase + c * chunk_s + j],
                    wsems[b],
                )

        def waitw(b):
            for j in range(chunk_s):
                pltpu.make_async_copy(
                    bufs[b].at[pl.ds(j * S1, S1)], out_hbm.at[sbase], wsems[b]
                ).wait()

        for b in range(_NBUF - 1):
            startg(b, b)
        waitg(0)
        startw(0, 0)
        startg(_NBUF - 1, _NBUF - 1)
        for c in range(1, _NBUF):
            b = c % _NBUF
            waitg(b)
            startw(c, b)
            waitw((b - 1) % _NBUF)
            startg(c + _NBUF - 1, (b - 1) % _NBUF)

        @pl.loop(1, n_outer - 1)
        def _steady(o):
            c0 = o * _NBUF
            for b in range(_NBUF):
                waitg(b)
                startw(c0 + b, b)
                waitw((b - 1) % _NBUF)
                startg(c0 + b + _NBUF - 1, (b - 1) % _NBUF)

        c0 = (n_outer - 1) * _NBUF
        for b in range(_NBUF):
            waitg(b)
            startw(c0 + b, b)
            waitw((b - 1) % _NBUF)
            if b == 0:
                startg(c0 + b + _NBUF - 1, (b - 1) % _NBUF)
        waitw(_NBUF - 1)

    return gather_kernel


def kernel(token_ids, weight):
    S0, S1 = token_ids.shape
    V, D = weight.shape
    idx = token_ids.reshape(-1).astype(jnp.int32)
    return _make_gather(S0, S1, V, D)(idx, weight)
